# baseline (speedup 1.0000x reference)
import functools

import jax
import jax.numpy as jnp
from jax import lax
from jax.experimental import pallas as pl
from jax.experimental.pallas import tpu as pltpu
from jax.experimental.pallas import tpu_sc as plsc

_L = 16


def _pos_embed_sc(row_embed, col_embed, B, H, W, D):
    C = 2 * D
    NW = 32
    ROWS = C // NW
    HW = H * W
    NROW, DROW = row_embed.shape
    NCOL, DCOL = col_embed.shape
    mesh = plsc.VectorSubcoreMesh(core_axis_name="c", subcore_axis_name="s")

    @functools.partial(
        pl.kernel,
        mesh=mesh,
        out_type=jax.ShapeDtypeStruct((B, C * HW), jnp.float32),
        scratch_types=[
            pltpu.VMEM((NROW * DROW,), jnp.float32),
            pltpu.VMEM((NCOL * DCOL,), jnp.float32),
            pltpu.VMEM((ROWS * HW,), jnp.float32),
            pltpu.SemaphoreType.DMA,
        ],
        compiler_params=pltpu.CompilerParams(needs_layout_passes=False),
    )
    def k(row_hbm, col_hbm, out_hbm, row_v, col_v, chunk, sem):
        cid = lax.axis_index("c")
        sid = lax.axis_index("s")
        wid = sid * 2 + cid
        base_c = wid * ROWS

        pltpu.sync_copy(row_hbm, row_v)
        pltpu.sync_copy(col_hbm, col_v)

        iota = lax.iota(jnp.int32, _L)

        @pl.when(base_c < D)
        def _col_half():
            for r in range(ROWS):
                vecs = [
                    plsc.load_gather(
                        col_v, [(iota + x0) * DCOL + (base_c + r)]
                    )
                    for x0 in range(0, W, _L)
                ]
                for y in range(H):
                    for i, v in enumerate(vecs):
                        chunk[pl.ds(r * HW + y * W + i * _L, _L)] = v

        @pl.when(base_c >= D)
        def _row_half():
            for r in range(ROWS):
                ec = base_c - D + r
                for y in range(H):
                    v = plsc.load_gather(
                        row_v, [jnp.full((_L,), y * DROW + ec, jnp.int32)]
                    )
                    for x0 in range(0, W, _L):
                        chunk[pl.ds(r * HW + y * W + x0, _L)] = v

        copies = [
            pltpu.async_copy(
                chunk, out_hbm.at[b, pl.ds(base_c * HW, ROWS * HW)], sem
            )
            for b in range(B)
        ]
        for cp in copies:
            cp.wait()

    return k(row_embed.reshape(-1), col_embed.reshape(-1))


def kernel(x, row_embed, col_embed):
    B = x.shape[0]
    H, W = x.shape[-2], x.shape[-1]
    D = row_embed.shape[-1]
    out = _pos_embed_sc(row_embed, col_embed, B, H, W, D)
    return out.reshape(B, 2 * D, H, W)


# speedup vs baseline: 3.3202x; 3.3202x over previous
import functools

import jax
import jax.numpy as jnp
from jax import lax
from jax.experimental import pallas as pl
from jax.experimental.pallas import tpu as pltpu
from jax.experimental.pallas import tpu_sc as plsc

_L = 16


def _pos_embed_sc(row_embed, col_embed, B, H, W, D):
    C = 2 * D
    NW = 32
    ROWS = C // NW
    HW = H * W
    NROW, DROW = row_embed.shape
    NCOL, DCOL = col_embed.shape
    mesh = plsc.VectorSubcoreMesh(core_axis_name="c", subcore_axis_name="s")

    @functools.partial(
        pl.kernel,
        mesh=mesh,
        out_type=jax.ShapeDtypeStruct((B, C, HW), jnp.float32),
        scratch_types=[
            pltpu.VMEM((NROW * DROW,), jnp.float32),
            pltpu.VMEM((NCOL * DCOL,), jnp.float32),
            pltpu.VMEM((ROWS, HW), jnp.float32),
            pltpu.SemaphoreType.DMA,
        ],
        compiler_params=pltpu.CompilerParams(needs_layout_passes=False),
    )
    def k(row_hbm, col_hbm, out_hbm, row_v, col_v, chunk, sem):
        cid = lax.axis_index("c")
        sid = lax.axis_index("s")
        wid = sid * 2 + cid
        base_c = wid * ROWS

        pltpu.sync_copy(row_hbm, row_v)
        pltpu.sync_copy(col_hbm, col_v)

        iota = lax.iota(jnp.int32, _L)

        @pl.when(base_c < D)
        def _col_half():
            for r in range(ROWS):
                vecs = [
                    plsc.load_gather(
                        col_v, [(iota + x0) * DCOL + (base_c + r)]
                    )
                    for x0 in range(0, W, _L)
                ]
                for y in range(H):
                    for i, v in enumerate(vecs):
                        chunk[r, pl.ds(y * W + i * _L, _L)] = v

        @pl.when(base_c >= D)
        def _row_half():
            for r in range(ROWS):
                ec = base_c - D + r
                for y in range(H):
                    v = plsc.load_gather(
                        row_v, [jnp.full((_L,), y * DROW + ec, jnp.int32)]
                    )
                    for x0 in range(0, W, _L):
                        chunk[r, pl.ds(y * W + x0, _L)] = v

        copies = [
            pltpu.async_copy(
                chunk, out_hbm.at[b, pl.ds(base_c, ROWS)], sem
            )
            for b in range(B)
        ]
        for cp in copies:
            cp.wait()

    return k(row_embed.reshape(-1), col_embed.reshape(-1))


def kernel(x, row_embed, col_embed):
    B = x.shape[0]
    H, W = x.shape[-2], x.shape[-1]
    D = row_embed.shape[-1]
    out = _pos_embed_sc(row_embed, col_embed, B, H, W, D)
    return out.reshape(B, 2 * D, H, W)


# speedup vs baseline: 3.3286x; 1.0026x over previous
import functools

import jax
import jax.numpy as jnp
from jax import lax
from jax.experimental import pallas as pl
from jax.experimental.pallas import tpu as pltpu
from jax.experimental.pallas import tpu_sc as plsc

_L = 16


def _pos_embed_sc(row_embed, col_embed, B, H, W, D):
    C = 2 * D
    NW = 32
    ROWS = C // NW
    HW = H * W
    NROW, DROW = row_embed.shape
    NCOL, DCOL = col_embed.shape
    mesh = plsc.VectorSubcoreMesh(core_axis_name="c", subcore_axis_name="s")

    @functools.partial(
        pl.kernel,
        mesh=mesh,
        out_type=jax.ShapeDtypeStruct((B, C, HW), jnp.float32),
        scratch_types=[
            pltpu.VMEM((NROW * DROW,), jnp.float32),
            pltpu.VMEM((NCOL * DCOL,), jnp.float32),
            pltpu.VMEM((ROWS, HW), jnp.float32),
            pltpu.SemaphoreType.DMA,
        ],
        compiler_params=pltpu.CompilerParams(
            needs_layout_passes=False, skip_device_barrier=True
        ),
    )
    def k(row_hbm, col_hbm, out_hbm, row_v, col_v, chunk, sem):
        cid = lax.axis_index("c")
        sid = lax.axis_index("s")
        wid = sid * 2 + cid
        base_c = wid * ROWS

        pltpu.sync_copy(row_hbm, row_v)
        pltpu.sync_copy(col_hbm, col_v)

        iota = lax.iota(jnp.int32, _L)

        @pl.when(base_c < D)
        def _col_half():
            for r in range(ROWS):
                vecs = [
                    plsc.load_gather(
                        col_v, [(iota + x0) * DCOL + (base_c + r)]
                    )
                    for x0 in range(0, W, _L)
                ]
                for y in range(H):
                    for i, v in enumerate(vecs):
                        chunk[r, pl.ds(y * W + i * _L, _L)] = v

        @pl.when(base_c >= D)
        def _row_half():
            for r in range(ROWS):
                ec = base_c - D + r
                for y in range(H):
                    v = plsc.load_gather(
                        row_v, [jnp.full((_L,), y * DROW + ec, jnp.int32)]
                    )
                    for x0 in range(0, W, _L):
                        chunk[r, pl.ds(y * W + x0, _L)] = v

        copies = [
            pltpu.async_copy(
                chunk, out_hbm.at[b, pl.ds(base_c, ROWS)], sem
            )
            for b in range(B)
        ]
        for cp in copies:
            cp.wait()

    return k(row_embed.reshape(-1), col_embed.reshape(-1))


def kernel(x, row_embed, col_embed):
    B = x.shape[0]
    H, W = x.shape[-2], x.shape[-1]
    D = row_embed.shape[-1]
    out = _pos_embed_sc(row_embed, col_embed, B, H, W, D)
    return out.reshape(B, 2 * D, H, W)
